# Initial kernel scaffold; baseline (speedup 1.0000x reference)
import jax
import jax.numpy as jnp
from jax.experimental import pallas as pl


def kernel(sparse_codes, edge_index, pattern_weights):
    raise NotImplementedError("write your pallas kernel here")


# speedup vs baseline: 8.1134x; 8.1134x over previous
import functools

import jax
import jax.numpy as jnp
from jax import lax
from jax.experimental import pallas as pl
from jax.experimental.pallas import tpu as pltpu
from jax.experimental.pallas import tpu_sc as plsc

N_NODES = 10000
N_EDGES = 320000
A = 128
L = 16
NC = 2
NS = 16
NW = NC * NS
E_PER = N_EDGES // NW
K = 80
NBLK = E_PER // K
NGRP = K // L
NJ = A // L


def _body(codes_hbm, sidx_hbm, didx_hbm, w_hbm, out_hbm,
          si_v, di_v, s0, s1, d0, d1, ost, wv, bt,
          ss0, ss1, ds0, ds1):
  wid = lax.axis_index("s") * NC + lax.axis_index("c")
  base = wid * E_PER

  pltpu.sync_copy(sidx_hbm.at[pl.ds(base, E_PER)], si_v)
  pltpu.sync_copy(didx_hbm.at[pl.ds(base, E_PER)], di_v)
  pltpu.sync_copy(w_hbm, wv)

  sbuf = (s0, s1)
  dbuf = (d0, d1)
  ssem = (ss0, ss1)
  dsem = (ds0, ds1)

  def start_blk(g, b):
    i0 = g * K
    pltpu.async_copy(codes_hbm.at[si_v.at[pl.ds(i0, K)]], sbuf[b], ssem[b])
    pltpu.async_copy(codes_hbm.at[di_v.at[pl.ds(i0, K)]], dbuf[b], dsem[b])

  def wait_blk(g, b):
    i0 = g * K
    pltpu.make_async_copy(
        codes_hbm.at[si_v.at[pl.ds(i0, K)]], sbuf[b], ssem[b]).wait()
    pltpu.make_async_copy(
        codes_hbm.at[di_v.at[pl.ds(i0, K)]], dbuf[b], dsem[b]).wait()

  lane = lax.iota(jnp.int32, L)

  def compute_blk(g, b):
    srows = sbuf[b]
    drows = dbuf[b]

    @pl.loop(0, NGRP)
    def _grp(grp):
      wregs = [wv[pl.ds(j * L, L)] for j in range(NJ)]
      e0 = g * K + grp * L
      for k in range(L):
        e = grp * L + k
        acc = srows[e, pl.ds(0, L)] * drows[e, pl.ds(0, L)] * wregs[0]
        for j in range(1, NJ):
          acc = jnp.maximum(
              acc,
              srows[e, pl.ds(j * L, L)] * drows[e, pl.ds(j * L, L)]
              * wregs[j])
        plsc.store_scatter(bt, [lane * L + k], acc)
      res = bt[pl.ds(0, L)]
      for l in range(1, L):
        res = jnp.maximum(res, bt[pl.ds(l * L, L)])
      ost[pl.ds(e0, L)] = res

  start_blk(0, 0)
  start_blk(1, 1)

  @pl.loop(0, (NBLK + 1) // 2)
  def _outer(gg):
    for b in range(2):
      g = gg * 2 + b

      @pl.when(g < NBLK)
      def _():
        wait_blk(g, b)

        @pl.when(g + 2 < NBLK)
        def _():
          start_blk(g + 2, b)

        compute_blk(g, b)

  @pl.loop(0, E_PER // L)
  def _sig(i):
    x = ost[pl.ds(i * L, L)]
    ost[pl.ds(i * L, L)] = 1.0 / (1.0 + jnp.exp(-x))

  pltpu.sync_copy(ost, out_hbm.at[pl.ds(base, E_PER)])


@jax.jit
def _run(codes, sidx, didx, w):
  mesh = plsc.VectorSubcoreMesh(
      core_axis_name="c", subcore_axis_name="s", num_cores=NC,
      num_subcores=NS)
  f = pl.kernel(
      _body,
      out_type=jax.ShapeDtypeStruct((N_EDGES,), jnp.float32),
      mesh=mesh,
      compiler_params=pltpu.CompilerParams(needs_layout_passes=False),
      scratch_types=[
          pltpu.VMEM((E_PER,), jnp.int32),
          pltpu.VMEM((E_PER,), jnp.int32),
          pltpu.VMEM((K, A), jnp.float32),
          pltpu.VMEM((K, A), jnp.float32),
          pltpu.VMEM((K, A), jnp.float32),
          pltpu.VMEM((K, A), jnp.float32),
          pltpu.VMEM((E_PER,), jnp.float32),
          pltpu.VMEM((A,), jnp.float32),
          pltpu.VMEM((L * L,), jnp.float32),
          pltpu.SemaphoreType.DMA,
          pltpu.SemaphoreType.DMA,
          pltpu.SemaphoreType.DMA,
          pltpu.SemaphoreType.DMA,
      ],
  )
  return f(codes, sidx, didx, w)


def kernel(sparse_codes, edge_index, pattern_weights):
  eidx = edge_index.astype(jnp.int32)
  return _run(sparse_codes, eidx[0], eidx[1], pattern_weights)
